# baseline (speedup 1.0000x reference)
import functools

import jax
import jax.numpy as jnp
from jax.experimental import pallas as pl
from jax.experimental.pallas import tpu as pltpu
from jax.experimental.pallas import tpu_sc as plsc

NUM_ROWS = 15
DIM = 32

_mesh = plsc.VectorSubcoreMesh(core_axis_name="c", subcore_axis_name="s", num_cores=1)


@functools.partial(
    pl.kernel,
    out_type=jax.ShapeDtypeStruct((1, DIM), jnp.float32),
    mesh=_mesh,
    scratch_types=[
        pltpu.VMEM((16,), jnp.int32),
    ],
)
def _lookup(table_hbm, id_hbm, out_hbm, idx_v):
    s = jax.lax.axis_index("s")

    @pl.when(s == 0)
    def _():
        pltpu.sync_copy(id_hbm, idx_v.at[pl.ds(0, 1)])
        i = idx_v[...][0]
        pltpu.sync_copy(table_hbm.at[pl.ds(i, 1)], out_hbm)


def kernel(table, decision_id):
    out = _lookup(table, decision_id.reshape(1).astype(jnp.int32))
    return out.reshape(DIM)


# speedup vs baseline: 1.0950x; 1.0950x over previous
import functools

import jax
import jax.numpy as jnp
from jax.experimental import pallas as pl
from jax.experimental.pallas import tpu as pltpu
from jax.experimental.pallas import tpu_sc as plsc

NUM_ROWS = 15
DIM = 32

_mesh = plsc.ScalarSubcoreMesh(axis_name="c", num_cores=1)


@functools.partial(
    pl.kernel,
    out_type=jax.ShapeDtypeStruct((1, DIM), jnp.float32),
    mesh=_mesh,
    scratch_types=[
        pltpu.SMEM((1,), jnp.int32),
    ],
)
def _lookup(table_hbm, id_hbm, out_hbm, id_s):
    pltpu.sync_copy(id_hbm, id_s)
    i = id_s[0]
    pltpu.sync_copy(table_hbm.at[pl.ds(i, 1)], out_hbm)


def kernel(table, decision_id):
    out = _lookup(table, decision_id.reshape(1).astype(jnp.int32))
    return out.reshape(DIM)
